# baseline (speedup 1.0000x reference)
import functools

import jax
import jax.numpy as jnp
from jax import lax
from jax.experimental import pallas as pl
from jax.experimental.pallas import tpu as pltpu
from jax.experimental.pallas import tpu_sc as plsc

BATCH = 16384
VOCAB = 100000
EMBED_DIM = 64
N_FEATS = 32
XW = 1 + N_FEATS
OUT_W = EMBED_DIM + N_FEATS
NC, NS, L = 2, 16, 16
NW = NC * NS
BPW = BATCH // NW
CHUNK = 128
NCHUNK = BPW // CHUNK
NBUF = 2


def kernel(x, W):
  mesh = plsc.VectorSubcoreMesh(
      core_axis_name="c", subcore_axis_name="s", num_cores=NC, num_subcores=NS
  )

  @functools.partial(
      pl.kernel,
      out_type=jax.ShapeDtypeStruct((BATCH, OUT_W), jnp.float32),
      mesh=mesh,
      scratch_types=[
          pltpu.VMEM((BPW * XW,), jnp.float32),
          pltpu.VMEM((BPW,), jnp.int32),
          pltpu.VMEM((NCHUNK, CHUNK), jnp.int32),
          pltpu.VMEM((NBUF, CHUNK, 2 * EMBED_DIM), jnp.float32),
          pltpu.VMEM((BPW, OUT_W), jnp.float32),
          pltpu.SemaphoreType.DMA,
          pltpu.SemaphoreType.DMA,
      ],
      compiler_params=pltpu.CompilerParams(needs_layout_passes=False),
  )
  def k(x_hbm, w_hbm, out_hbm, x_v, ids_v, idx_v, buf_v, out_v, sem_a, sem_b):
    sems = [sem_a, sem_b]
    wid = lax.axis_index("s") * NC + lax.axis_index("c")
    base = wid * BPW
    pltpu.sync_copy(x_hbm.at[pl.ds(base * XW, BPW * XW)], x_v)
    for i in range(BPW // L):
      flat = (lax.iota(jnp.int32, L) + i * L) * XW
      ids = plsc.load_gather(x_v, [flat]).astype(jnp.int32)
      ids_v[pl.ds(i * L, L)] = ids
      idx_v[(i * L) // CHUNK, pl.ds((i * L) % CHUNK, L)] = (
          lax.shift_right_logical(ids, 1))

    def fire(j):
      return pltpu.async_copy(
          w_hbm.at[idx_v.at[j]], buf_v.at[j % NBUF], sems[j % NBUF])

    pending = {}
    for j in range(min(NBUF, NCHUNK)):
      pending[j] = fire(j)
    for j in range(NCHUNK):
      pending.pop(j).wait()

      def body(g, _):
        rows = j * CHUNK + g * L
        idv = ids_v[pl.ds(rows, L)] & 1
        for t in range(L):
          r = rows + t
          off = idv[t] * EMBED_DIM
          for c in range(EMBED_DIM // L):
            out_v[r, pl.ds(c * L, L)] = buf_v[j % NBUF, g * L + t,
                                              pl.ds(off + c * L, L)]
          out_v[r, pl.ds(EMBED_DIM, L)] = x_v[pl.ds(r * XW + 1, L)]
          out_v[r, pl.ds(EMBED_DIM + L, L)] = x_v[pl.ds(r * XW + 1 + L, L)]
        return 0

      lax.fori_loop(0, CHUNK // L, body, 0)
      if j + NBUF < NCHUNK:
        pending[j + NBUF] = fire(j + NBUF)
    pltpu.sync_copy(out_v, out_hbm.at[pl.ds(base, BPW)])

  return k(x.reshape(-1), W.reshape(VOCAB // 2, 2 * EMBED_DIM))


# speedup vs baseline: 1.8930x; 1.8930x over previous
import functools

import jax
import jax.numpy as jnp
from jax import lax
from jax.experimental import pallas as pl
from jax.experimental.pallas import tpu as pltpu
from jax.experimental.pallas import tpu_sc as plsc

BATCH = 16384
VOCAB = 100000
EMBED_DIM = 64
N_FEATS = 32
OUT_W = EMBED_DIM + N_FEATS
NC, NS, L = 2, 16, 16
NW = NC * NS
DPW = EMBED_DIM // NW
OCH = 8192


def kernel(x, W):
  mesh = plsc.VectorSubcoreMesh(
      core_axis_name="c", subcore_axis_name="s", num_cores=NC, num_subcores=NS
  )

  @functools.partial(
      pl.kernel,
      out_type=jax.ShapeDtypeStruct((OUT_W, BATCH), jnp.float32),
      mesh=mesh,
      scratch_types=[
          pltpu.VMEM((VOCAB,), jnp.float32),
          pltpu.VMEM((BATCH,), jnp.int32),
          pltpu.VMEM((OCH,), jnp.float32),
          pltpu.SemaphoreType.DMA,
      ],
      compiler_params=pltpu.CompilerParams(needs_layout_passes=False),
  )
  def k(xt_hbm, wt_hbm, out_hbm, w_row, ids_v, ob_v, sem):
    wid = lax.axis_index("s") * NC + lax.axis_index("c")

    for h in range(BATCH // OCH):
      pltpu.sync_copy(xt_hbm.at[0, pl.ds(h * OCH, OCH)], ob_v)

      def conv(i, _):
        ids_v[pl.ds(h * OCH + i * L, L)] = (
            ob_v[pl.ds(i * L, L)].astype(jnp.int32))
        return 0

      lax.fori_loop(0, OCH // L, conv, 0)

    for h in range(BATCH // OCH):
      pltpu.sync_copy(xt_hbm.at[1 + wid, pl.ds(h * OCH, OCH)], ob_v)
      pltpu.sync_copy(ob_v, out_hbm.at[EMBED_DIM + wid, pl.ds(h * OCH, OCH)])

    for t in range(DPW):
      d = wid * DPW + t
      pltpu.async_copy(wt_hbm.at[d], w_row, sem).wait()
      for h in range(BATCH // OCH):

        def body(i, _):
          idx = ids_v[pl.ds(h * OCH + i * L, L)]
          ob_v[pl.ds(i * L, L)] = plsc.load_gather(w_row, [idx])
          return 0

        lax.fori_loop(0, OCH // L, body, 0)
        pltpu.sync_copy(ob_v, out_hbm.at[d, pl.ds(h * OCH, OCH)])

  out_t = k(x.T, W.T)
  return out_t.T


# speedup vs baseline: 2.6872x; 1.4195x over previous
import functools

import jax
import jax.numpy as jnp
from jax import lax
from jax.experimental import pallas as pl
from jax.experimental.pallas import tpu as pltpu
from jax.experimental.pallas import tpu_sc as plsc

BATCH = 16384
VOCAB = 100000
EMBED_DIM = 64
N_FEATS = 32
OUT_W = EMBED_DIM + N_FEATS
NC, NS, L = 2, 16, 16
NW = NC * NS
DPW = EMBED_DIM // NW
OCH = 4096
NCH = BATCH // OCH
UNROLL = 8


def kernel(x, W):
  mesh = plsc.VectorSubcoreMesh(
      core_axis_name="c", subcore_axis_name="s", num_cores=NC, num_subcores=NS
  )

  @functools.partial(
      pl.kernel,
      out_type=jax.ShapeDtypeStruct((OUT_W, BATCH), jnp.float32),
      mesh=mesh,
      scratch_types=[
          pltpu.VMEM((VOCAB,), jnp.float32),
          pltpu.VMEM((BATCH,), jnp.int32),
          pltpu.VMEM((2, OCH), jnp.float32),
          pltpu.SemaphoreType.DMA,
          pltpu.SemaphoreType.DMA,
          pltpu.SemaphoreType.DMA,
          pltpu.SemaphoreType.DMA,
          pltpu.SemaphoreType.DMA,
      ],
      compiler_params=pltpu.CompilerParams(needs_layout_passes=False),
  )
  def k(xt_hbm, wt_hbm, out_hbm, w_row, ids_v, ob_v,
        sem_w, sem_ra, sem_rb, sem_wa, sem_wb):
    rsems = [sem_ra, sem_rb]
    wsems = [sem_wa, sem_wb]
    wid = lax.axis_index("s") * NC + lax.axis_index("c")
    d0 = wid * DPW

    w_copy = pltpu.async_copy(wt_hbm.at[d0], w_row, sem_w)

    reads = {}
    for h in range(min(2, NCH)):
      reads[h] = pltpu.async_copy(
          xt_hbm.at[0, pl.ds(h * OCH, OCH)], ob_v.at[h % 2], rsems[h % 2])
    for h in range(NCH):
      reads.pop(h).wait()

      @plsc.parallel_loop(0, OCH, step=L, unroll=UNROLL)
      def conv(i):
        ids_v[pl.ds(h * OCH + i, L)] = (
            ob_v[h % 2, pl.ds(i, L)].astype(jnp.int32))

      if h + 2 < NCH:
        reads[h + 2] = pltpu.async_copy(
            xt_hbm.at[0, pl.ds((h + 2) * OCH, OCH)], ob_v.at[h % 2],
            rsems[h % 2])

    writes = {}
    for h in range(NCH):
      pltpu.async_copy(
          xt_hbm.at[1 + wid, pl.ds(h * OCH, OCH)], ob_v.at[h % 2],
          rsems[h % 2]).wait()
      writes[h] = pltpu.async_copy(
          ob_v.at[h % 2], out_hbm.at[EMBED_DIM + wid, pl.ds(h * OCH, OCH)],
          wsems[h % 2])
      if h >= 1:
        writes.pop(h - 1).wait()
    writes.pop(NCH - 1).wait()

    for t in range(DPW):
      d = d0 + t
      w_copy.wait()
      for h in range(NCH):
        if h >= 2:
          writes.pop(h - 2).wait()

        @plsc.parallel_loop(0, OCH, step=L, unroll=UNROLL)
        def body(i):
          idx = ids_v[pl.ds(h * OCH + i, L)]
          ob_v[h % 2, pl.ds(i, L)] = plsc.load_gather(w_row, [idx])

        if h == NCH - 1 and t + 1 < DPW:
          w_copy = pltpu.async_copy(wt_hbm.at[d + 1], w_row, sem_w)
        writes[h] = pltpu.async_copy(
            ob_v.at[h % 2], out_hbm.at[d, pl.ds(h * OCH, OCH)], wsems[h % 2])
      writes.pop(NCH - 2).wait()
      writes.pop(NCH - 1).wait()

  out_t = k(x.T, W.T)
  return out_t.T
